# baseline (speedup 1.0000x reference)
import functools

import jax
import jax.numpy as jnp
from jax import lax
from jax.experimental import pallas as pl
from jax.experimental.pallas import tpu as pltpu
from jax.experimental.pallas import tpu_sc as plsc

N = 10000
F = 128
E = 320000
LEVELS = 4

NC = 2
NS = 16

HALF = N // NC
C = 128
PER_TILE = E // NS
NVEC = PER_TILE // 16
ACC_N = 5376
DUMMY0 = 5120
INIT_A = 320
INIT_B = HALF - 15 * INIT_A
ZPAD = ACC_N - HALF

NB = 5
BLK = N // NB


def _transform_body(x_ref, lv_ref, w_ref, b_ref, y_ref):
    x = x_ref[...]
    lv = lv_ref[0, 0, :]
    acc = jnp.zeros_like(x)
    for l in range(LEVELS):
        m = (lv == l).astype(jnp.float32)[:, None]
        xw = lax.dot_general(x * m, w_ref[l], (((1,), (1,)), ((), ())),
                             preferred_element_type=jnp.float32)
        acc = acc + xw + m * b_ref[l][None, :]
    y_ref[...] = 0.2 * acc


def _sc_body(y_hbm, src_hbm, dst_hbm, lv_hbm, x_hbm, zin_hbm, out_hbm,
             lv_v, gsrc_v, gdst_v,
             sidx_a, sidx_b, rows_a, rows_b, acc_sh, sem_a, sem_b):
    c = lax.axis_index("c")
    s = lax.axis_index("s")
    base = c * HALF

    @pl.when(s < NS - 1)
    def _initx_a():
        pltpu.sync_copy(x_hbm.at[pl.ds(base + s * INIT_A, INIT_A)],
                        acc_sh.at[pl.ds(s * INIT_A, INIT_A)])

    @pl.when(s == NS - 1)
    def _initx_b():
        pltpu.sync_copy(x_hbm.at[pl.ds(base + 15 * INIT_A, INIT_B)],
                        acc_sh.at[pl.ds(15 * INIT_A, INIT_B)])
        pltpu.sync_copy(zin_hbm, acc_sh.at[pl.ds(HALF, ZPAD)])

    pltpu.sync_copy(lv_hbm, lv_v)
    pltpu.sync_copy(src_hbm.at[pl.ds(s * PER_TILE, PER_TILE)],
                    gsrc_v.at[pl.ds(0, PER_TILE)])
    pltpu.sync_copy(dst_hbm.at[pl.ds(s * PER_TILE, PER_TILE)],
                    gdst_v.at[pl.ds(0, PER_TILE)])
    plsc.subcore_barrier()

    def cvec(v, cnt):
        srcs = gsrc_v[pl.ds(v * 16, 16)]
        dsts = gdst_v[pl.ds(v * 16, 16)]
        ll = plsc.load_gather(lv_v, [srcs])
        hl = plsc.load_gather(lv_v, [dsts])
        local = dsts - base
        valid = ((ll >= 0) & (ll < LEVELS) & (hl > ll)
                 & (local >= 0) & (local < HALF))
        plsc.store_compressed(gsrc_v.at[pl.ds(cnt, 16)], srcs, mask=valid)
        plsc.store_compressed(gdst_v.at[pl.ds(cnt, 16)], local, mask=valid)
        return cnt + plsc.all_reduce_population_count(valid)[0]

    cnt = lax.fori_loop(0, NVEC, cvec, jnp.int32(0))

    for v in range(C // 16):
        dummy = DUMMY0 + v * 16 + lax.iota(jnp.int32, 16)
        gsrc_v[pl.ds(cnt + v * 16, 16)] = jnp.zeros((16,), jnp.int32)
        gdst_v[pl.ds(cnt + v * 16, 16)] = dummy

    nch = (cnt + C - 1) // C

    def fill_sidx(j, sidx):
        for v in range(C // 16):
            sidx[pl.ds(v * 16, 16)] = gdst_v[pl.ds(j * C + v * 16, 16)]

    def start_gather(j, rows, sem):
        return pltpu.async_copy(y_hbm.at[gsrc_v.at[pl.ds(j * C, C)]], rows, sem)

    @pl.when(nch > 0)
    def _prologue():
        fill_sidx(0, sidx_a)
        start_gather(0, rows_a, sem_a)

    def pair(p, carry):
        j0 = 2 * p
        j1 = j0 + 1

        @pl.when(j1 < nch)
        def _startb():
            fill_sidx(j1, sidx_b)
            start_gather(j1, rows_b, sem_b)

        pltpu.make_async_copy(y_hbm.at[gsrc_v.at[pl.ds(0, C)]], rows_a, sem_a).wait()
        pltpu.sync_copy(rows_a, acc_sh.at[sidx_a], add=True)

        @pl.when(j0 + 2 < nch)
        def _starta():
            fill_sidx(j0 + 2, sidx_a)
            start_gather(j0 + 2, rows_a, sem_a)

        @pl.when(j1 < nch)
        def _drainb():
            pltpu.make_async_copy(y_hbm.at[gsrc_v.at[pl.ds(0, C)]], rows_b, sem_b).wait()
            pltpu.sync_copy(rows_b, acc_sh.at[sidx_b], add=True)

        return carry

    lax.fori_loop(0, (nch + 1) // 2, pair, jnp.int32(0))
    plsc.subcore_barrier()

    @pl.when(s < NS - 1)
    def _out_a():
        pltpu.sync_copy(acc_sh.at[pl.ds(s * INIT_A, INIT_A)],
                        out_hbm.at[pl.ds(base + s * INIT_A, INIT_A)])

    @pl.when(s == NS - 1)
    def _out_b():
        pltpu.sync_copy(acc_sh.at[pl.ds(15 * INIT_A, INIT_B)],
                        out_hbm.at[pl.ds(base + 15 * INIT_A, INIT_B)])


_sc_edges = functools.partial(
    pl.kernel,
    out_type=jax.ShapeDtypeStruct((N, F), jnp.float32),
    mesh=plsc.VectorSubcoreMesh(core_axis_name="c", subcore_axis_name="s"),
    compiler_params=pltpu.CompilerParams(needs_layout_passes=False),
    scratch_types=[
        pltpu.VMEM((N,), jnp.int32),
        pltpu.VMEM((PER_TILE + C,), jnp.int32),
        pltpu.VMEM((PER_TILE + C,), jnp.int32),
        pltpu.VMEM((C,), jnp.int32),
        pltpu.VMEM((C,), jnp.int32),
        pltpu.VMEM((C, F), jnp.float32),
        pltpu.VMEM((C, F), jnp.float32),
        pltpu.VMEM_SHARED((ACC_N, F), jnp.float32),
        pltpu.SemaphoreType.DMA,
        pltpu.SemaphoreType.DMA,
    ],
)(_sc_body)


def kernel(node_features, hierarchy_edges, hierarchy_levels, level_weights, level_biases):
    src_p = hierarchy_edges[:, 0]
    dst_p = hierarchy_edges[:, 1]
    lv3 = hierarchy_levels.reshape(NB, 1, BLK)

    y = pl.pallas_call(
        _transform_body,
        grid=(NB,),
        in_specs=[
            pl.BlockSpec((BLK, F), lambda i: (i, 0)),
            pl.BlockSpec((1, 1, BLK), lambda i: (i, 0, 0)),
            pl.BlockSpec((LEVELS, F, F), lambda i: (0, 0, 0)),
            pl.BlockSpec((LEVELS, F), lambda i: (0, 0)),
        ],
        out_specs=pl.BlockSpec((BLK, F), lambda i: (i, 0)),
        out_shape=jax.ShapeDtypeStruct((N, F), jnp.float32),
    )(node_features, lv3, level_weights, level_biases)

    zin = jnp.zeros((ZPAD, F), jnp.float32)
    out = _sc_edges(y, src_p, dst_p, hierarchy_levels, node_features, zin)
    return out


# speedup vs baseline: 1.1663x; 1.1663x over previous
import functools

import jax
import jax.numpy as jnp
from jax import lax
from jax.experimental import pallas as pl
from jax.experimental.pallas import tpu as pltpu
from jax.experimental.pallas import tpu_sc as plsc

N = 10000
F = 128
E = 320000
LEVELS = 4

NC = 2
NS = 16
NW = NC * NS

C = 64
PER_TILE = E // NW
SB = 2000
NSB = PER_TILE // SB
SBU = 5
ACC_N = 10240
ROWS_PER_TILE = ACC_N // NS
DUMMY0 = N

NB = 5
BLK = N // NB


def _transform_body(x_ref, lv_ref, w_ref, b_ref, y_ref):
    x = x_ref[...]
    lv = lv_ref[0, 0, :]
    acc = jnp.zeros_like(x)
    for l in range(LEVELS):
        m = (lv == l).astype(jnp.float32)[:, None]
        xw = lax.dot_general(x * m, w_ref[l], (((1,), (1,)), ((), ())),
                             preferred_element_type=jnp.float32)
        acc = acc + xw + m * b_ref[l][None, :]
    y_ref[...] = 0.2 * acc


def _merge_body(x_ref, a_ref, o_ref):
    o_ref[...] = x_ref[...] + a_ref[0] + a_ref[1]


def _sc_body(y_hbm, src_hbm, dst_hbm, lv_hbm, zin_hbm, out_hbm,
             lv_v, ssrc_a, sdst_a, ssrc_b, sdst_b, gcomb_v,
             gidx_a, gidx_b, sidx_a, sidx_b, rows_a, rows_b, acc_sh,
             sem_sa, sem_sb, sem_a, sem_b):
    c = lax.axis_index("c")
    s = lax.axis_index("s")
    wid = s * NC + c
    ebase = wid * PER_TILE

    pltpu.sync_copy(zin_hbm, acc_sh.at[pl.ds(s * ROWS_PER_TILE, ROWS_PER_TILE)])
    pltpu.sync_copy(lv_hbm, lv_v)

    pltpu.async_copy(src_hbm.at[pl.ds(ebase, SB)], ssrc_a, sem_sa)
    pltpu.async_copy(dst_hbm.at[pl.ds(ebase, SB)], sdst_a, sem_sa)
    plsc.subcore_barrier()

    def compact_block(ssrc, sdst, cnt):
        def cvec(v, cnt):
            for u in range(SBU):
                off = (v * SBU + u) * 16
                srcs = ssrc[pl.ds(off, 16)]
                dsts = sdst[pl.ds(off, 16)]
                ll = plsc.load_gather(lv_v, [srcs])
                hl = plsc.load_gather(lv_v, [dsts])
                valid = (ll >= 0) & (ll < LEVELS) & (hl > ll)
                packed = lax.shift_left(srcs, 16) | dsts
                plsc.store_compressed(gcomb_v.at[pl.ds(cnt, 16)], packed,
                                      mask=valid)
                cnt = cnt + plsc.all_reduce_population_count(valid)[0]
            return cnt
        return lax.fori_loop(0, SB // (16 * SBU), cvec, cnt)

    cnt = jnp.int32(0)
    for b in range(NSB):
        cur_src, cur_dst = (ssrc_a, sdst_a) if b % 2 == 0 else (ssrc_b, sdst_b)
        cur_sem = sem_sa if b % 2 == 0 else sem_sb
        nxt_src, nxt_dst = (ssrc_b, sdst_b) if b % 2 == 0 else (ssrc_a, sdst_a)
        nxt_sem = sem_sb if b % 2 == 0 else sem_sa
        if b + 1 < NSB:
            pltpu.async_copy(src_hbm.at[pl.ds(ebase + (b + 1) * SB, SB)],
                             nxt_src, nxt_sem)
            pltpu.async_copy(dst_hbm.at[pl.ds(ebase + (b + 1) * SB, SB)],
                             nxt_dst, nxt_sem)
        pltpu.make_async_copy(src_hbm.at[pl.ds(0, SB)], cur_src, cur_sem).wait()
        pltpu.make_async_copy(dst_hbm.at[pl.ds(0, SB)], cur_dst, cur_sem).wait()
        cnt = compact_block(cur_src, cur_dst, cnt)

    for v in range(C // 16):
        dummy = DUMMY0 + v * 16 + lax.iota(jnp.int32, 16)
        gcomb_v[pl.ds(cnt + v * 16, 16)] = dummy

    nch = (cnt + C - 1) // C

    def fill_idx(j, gidx, sidx):
        for v in range(C // 16):
            packed = gcomb_v[pl.ds(j * C + v * 16, 16)]
            gidx[pl.ds(v * 16, 16)] = lax.shift_right_logical(packed, 16)
            sidx[pl.ds(v * 16, 16)] = packed & 0xFFFF

    def start_gather(rows, gidx, sem):
        return pltpu.async_copy(y_hbm.at[gidx], rows, sem)

    @pl.when(nch > 0)
    def _prologue():
        fill_idx(0, gidx_a, sidx_a)
        start_gather(rows_a, gidx_a, sem_a)

    def pair(p, carry):
        j0 = 2 * p
        j1 = j0 + 1

        @pl.when(j1 < nch)
        def _startb():
            fill_idx(j1, gidx_b, sidx_b)
            start_gather(rows_b, gidx_b, sem_b)

        pltpu.make_async_copy(y_hbm.at[gidx_a], rows_a, sem_a).wait()
        pltpu.sync_copy(rows_a, acc_sh.at[sidx_a], add=True)

        @pl.when(j0 + 2 < nch)
        def _starta():
            fill_idx(j0 + 2, gidx_a, sidx_a)
            start_gather(rows_a, gidx_a, sem_a)

        @pl.when(j1 < nch)
        def _drainb():
            pltpu.make_async_copy(y_hbm.at[gidx_b], rows_b, sem_b).wait()
            pltpu.sync_copy(rows_b, acc_sh.at[sidx_b], add=True)

        return carry

    lax.fori_loop(0, (nch + 1) // 2, pair, jnp.int32(0))
    plsc.subcore_barrier()

    pltpu.sync_copy(acc_sh.at[pl.ds(s * ROWS_PER_TILE, ROWS_PER_TILE)],
                    out_hbm.at[c, pl.ds(s * ROWS_PER_TILE, ROWS_PER_TILE)])


_sc_edges = functools.partial(
    pl.kernel,
    out_type=jax.ShapeDtypeStruct((NC, ACC_N, F), jnp.float32),
    mesh=plsc.VectorSubcoreMesh(core_axis_name="c", subcore_axis_name="s"),
    compiler_params=pltpu.CompilerParams(needs_layout_passes=False),
    scratch_types=[
        pltpu.VMEM((N,), jnp.int32),
        pltpu.VMEM((SB,), jnp.int32),
        pltpu.VMEM((SB,), jnp.int32),
        pltpu.VMEM((SB,), jnp.int32),
        pltpu.VMEM((SB,), jnp.int32),
        pltpu.VMEM((PER_TILE + C,), jnp.int32),
        pltpu.VMEM((C,), jnp.int32),
        pltpu.VMEM((C,), jnp.int32),
        pltpu.VMEM((C,), jnp.int32),
        pltpu.VMEM((C,), jnp.int32),
        pltpu.VMEM((C, F), jnp.float32),
        pltpu.VMEM((C, F), jnp.float32),
        pltpu.VMEM_SHARED((ACC_N, F), jnp.float32),
        pltpu.SemaphoreType.DMA,
        pltpu.SemaphoreType.DMA,
        pltpu.SemaphoreType.DMA,
        pltpu.SemaphoreType.DMA,
    ],
)(_sc_body)


def kernel(node_features, hierarchy_edges, hierarchy_levels, level_weights, level_biases):
    src_p = hierarchy_edges[:, 0]
    dst_p = hierarchy_edges[:, 1]
    lv3 = hierarchy_levels.reshape(NB, 1, BLK)

    y = pl.pallas_call(
        _transform_body,
        grid=(NB,),
        in_specs=[
            pl.BlockSpec((BLK, F), lambda i: (i, 0)),
            pl.BlockSpec((1, 1, BLK), lambda i: (i, 0, 0)),
            pl.BlockSpec((LEVELS, F, F), lambda i: (0, 0, 0)),
            pl.BlockSpec((LEVELS, F), lambda i: (0, 0)),
        ],
        out_specs=pl.BlockSpec((BLK, F), lambda i: (i, 0)),
        out_shape=jax.ShapeDtypeStruct((N, F), jnp.float32),
    )(node_features, lv3, level_weights, level_biases)

    zin = jnp.zeros((ROWS_PER_TILE, F), jnp.float32)
    parts = _sc_edges(y, src_p, dst_p, hierarchy_levels, zin)

    out = pl.pallas_call(
        _merge_body,
        grid=(NB,),
        in_specs=[
            pl.BlockSpec((BLK, F), lambda i: (i, 0)),
            pl.BlockSpec((NC, BLK, F), lambda i: (0, i, 0)),
        ],
        out_specs=pl.BlockSpec((BLK, F), lambda i: (i, 0)),
        out_shape=jax.ShapeDtypeStruct((N, F), jnp.float32),
    )(node_features, parts)
    return out


# speedup vs baseline: 1.1676x; 1.0011x over previous
import functools

import jax
import jax.numpy as jnp
from jax import lax
from jax.experimental import pallas as pl
from jax.experimental.pallas import tpu as pltpu
from jax.experimental.pallas import tpu_sc as plsc

N = 10000
F = 128
E = 320000
LEVELS = 4

NC = 2
NS = 16
NW = NC * NS

C = 64
PER_TILE = E // NW
SB = 2000
NSB = PER_TILE // SB
SBU = 5
ACC_N = 10240
ROWS_PER_TILE = ACC_N // NS
DUMMY0 = N

NB = 5
BLK = N // NB


def _transform_body(x_ref, lv_ref, w_ref, b_ref, y_ref):
    x = x_ref[...]
    lv = lv_ref[0, 0, :]
    acc = jnp.zeros_like(x)
    for l in range(LEVELS):
        m = (lv == l).astype(jnp.float32)[:, None]
        xw = lax.dot_general(x * m, w_ref[l], (((1,), (1,)), ((), ())),
                             preferred_element_type=jnp.float32)
        acc = acc + xw + m * b_ref[l][None, :]
    y_ref[...] = 0.2 * acc


def _merge_body(x_ref, a_ref, o_ref):
    o_ref[...] = x_ref[...] + a_ref[0] + a_ref[1]


def _sc_body(y_hbm, src_hbm, dst_hbm, lv_hbm, zin_hbm, out_hbm,
             lv_v, ssrc_a, sdst_a, ssrc_b, sdst_b, gcomb_v,
             gidx_a, gidx_b, sidx_a, sidx_b, rows_a, rows_b, acc_sh,
             sem_sa, sem_sb, sem_a, sem_b):
    c = lax.axis_index("c")
    s = lax.axis_index("s")
    wid = s * NC + c
    ebase = wid * PER_TILE

    pltpu.sync_copy(zin_hbm, acc_sh.at[pl.ds(s * ROWS_PER_TILE, ROWS_PER_TILE)])
    pltpu.sync_copy(lv_hbm, lv_v)

    pltpu.async_copy(src_hbm.at[pl.ds(ebase, SB)], ssrc_a, sem_sa)
    pltpu.async_copy(dst_hbm.at[pl.ds(ebase, SB)], sdst_a, sem_sa)
    plsc.subcore_barrier()

    def compact_block(ssrc, sdst, cnt):
        def cvec(v, cnt):
            for u in range(SBU):
                off = (v * SBU + u) * 16
                srcs = ssrc[pl.ds(off, 16)]
                dsts = sdst[pl.ds(off, 16)]
                ll = plsc.load_gather(lv_v, [srcs])
                hl = plsc.load_gather(lv_v, [dsts])
                valid = (ll >= 0) & (ll < LEVELS) & (hl > ll)
                packed = lax.shift_left(srcs, 16) | dsts
                plsc.store_compressed(gcomb_v.at[pl.ds(cnt, 16)], packed,
                                      mask=valid)
                cnt = cnt + plsc.all_reduce_population_count(valid)[0]
            return cnt
        return lax.fori_loop(0, SB // (16 * SBU), cvec, cnt)

    cnt = jnp.int32(0)
    scope_p1 = jax.named_scope("p1_compact")
    scope_p1.__enter__()
    for b in range(NSB):
        cur_src, cur_dst = (ssrc_a, sdst_a) if b % 2 == 0 else (ssrc_b, sdst_b)
        cur_sem = sem_sa if b % 2 == 0 else sem_sb
        nxt_src, nxt_dst = (ssrc_b, sdst_b) if b % 2 == 0 else (ssrc_a, sdst_a)
        nxt_sem = sem_sb if b % 2 == 0 else sem_sa
        if b + 1 < NSB:
            pltpu.async_copy(src_hbm.at[pl.ds(ebase + (b + 1) * SB, SB)],
                             nxt_src, nxt_sem)
            pltpu.async_copy(dst_hbm.at[pl.ds(ebase + (b + 1) * SB, SB)],
                             nxt_dst, nxt_sem)
        pltpu.make_async_copy(src_hbm.at[pl.ds(0, SB)], cur_src, cur_sem).wait()
        pltpu.make_async_copy(dst_hbm.at[pl.ds(0, SB)], cur_dst, cur_sem).wait()
        cnt = compact_block(cur_src, cur_dst, cnt)

    for v in range(C // 16):
        dummy = DUMMY0 + v * 16 + lax.iota(jnp.int32, 16)
        gcomb_v[pl.ds(cnt + v * 16, 16)] = dummy

    nch = (cnt + C - 1) // C
    scope_p1.__exit__(None, None, None)
    scope_p2 = jax.named_scope("p2_scatter")
    scope_p2.__enter__()

    def fill_idx(j, gidx, sidx):
        for v in range(C // 16):
            packed = gcomb_v[pl.ds(j * C + v * 16, 16)]
            gidx[pl.ds(v * 16, 16)] = lax.shift_right_logical(packed, 16)
            sidx[pl.ds(v * 16, 16)] = packed & 0xFFFF

    def start_gather(rows, gidx, sem):
        return pltpu.async_copy(y_hbm.at[gidx], rows, sem)

    @pl.when(nch > 0)
    def _prologue():
        fill_idx(0, gidx_a, sidx_a)
        start_gather(rows_a, gidx_a, sem_a)

    def pair(p, carry):
        j0 = 2 * p
        j1 = j0 + 1

        @pl.when(j1 < nch)
        def _startb():
            fill_idx(j1, gidx_b, sidx_b)
            start_gather(rows_b, gidx_b, sem_b)

        pltpu.make_async_copy(y_hbm.at[gidx_a], rows_a, sem_a).wait()
        pltpu.sync_copy(rows_a, acc_sh.at[sidx_a], add=True)

        @pl.when(j0 + 2 < nch)
        def _starta():
            fill_idx(j0 + 2, gidx_a, sidx_a)
            start_gather(rows_a, gidx_a, sem_a)

        @pl.when(j1 < nch)
        def _drainb():
            pltpu.make_async_copy(y_hbm.at[gidx_b], rows_b, sem_b).wait()
            pltpu.sync_copy(rows_b, acc_sh.at[sidx_b], add=True)

        return carry

    lax.fori_loop(0, (nch + 1) // 2, pair, jnp.int32(0))
    scope_p2.__exit__(None, None, None)
    plsc.subcore_barrier()

    pltpu.sync_copy(acc_sh.at[pl.ds(s * ROWS_PER_TILE, ROWS_PER_TILE)],
                    out_hbm.at[c, pl.ds(s * ROWS_PER_TILE, ROWS_PER_TILE)])


_sc_edges = functools.partial(
    pl.kernel,
    out_type=jax.ShapeDtypeStruct((NC, ACC_N, F), jnp.float32),
    mesh=plsc.VectorSubcoreMesh(core_axis_name="c", subcore_axis_name="s"),
    compiler_params=pltpu.CompilerParams(needs_layout_passes=False),
    scratch_types=[
        pltpu.VMEM((N,), jnp.int32),
        pltpu.VMEM((SB,), jnp.int32),
        pltpu.VMEM((SB,), jnp.int32),
        pltpu.VMEM((SB,), jnp.int32),
        pltpu.VMEM((SB,), jnp.int32),
        pltpu.VMEM((PER_TILE + C,), jnp.int32),
        pltpu.VMEM((C,), jnp.int32),
        pltpu.VMEM((C,), jnp.int32),
        pltpu.VMEM((C,), jnp.int32),
        pltpu.VMEM((C,), jnp.int32),
        pltpu.VMEM((C, F), jnp.float32),
        pltpu.VMEM((C, F), jnp.float32),
        pltpu.VMEM_SHARED((ACC_N, F), jnp.float32),
        pltpu.SemaphoreType.DMA,
        pltpu.SemaphoreType.DMA,
        pltpu.SemaphoreType.DMA,
        pltpu.SemaphoreType.DMA,
    ],
)(_sc_body)


def kernel(node_features, hierarchy_edges, hierarchy_levels, level_weights, level_biases):
    src_p = hierarchy_edges[:, 0]
    dst_p = hierarchy_edges[:, 1]
    lv3 = hierarchy_levels.reshape(NB, 1, BLK)

    y = pl.pallas_call(
        _transform_body,
        grid=(NB,),
        in_specs=[
            pl.BlockSpec((BLK, F), lambda i: (i, 0)),
            pl.BlockSpec((1, 1, BLK), lambda i: (i, 0, 0)),
            pl.BlockSpec((LEVELS, F, F), lambda i: (0, 0, 0)),
            pl.BlockSpec((LEVELS, F), lambda i: (0, 0)),
        ],
        out_specs=pl.BlockSpec((BLK, F), lambda i: (i, 0)),
        out_shape=jax.ShapeDtypeStruct((N, F), jnp.float32),
    )(node_features, lv3, level_weights, level_biases)

    zin = jnp.zeros((ROWS_PER_TILE, F), jnp.float32)
    parts = _sc_edges(y, src_p, dst_p, hierarchy_levels, zin)

    out = pl.pallas_call(
        _merge_body,
        grid=(NB,),
        in_specs=[
            pl.BlockSpec((BLK, F), lambda i: (i, 0)),
            pl.BlockSpec((NC, BLK, F), lambda i: (0, i, 0)),
        ],
        out_specs=pl.BlockSpec((BLK, F), lambda i: (i, 0)),
        out_shape=jax.ShapeDtypeStruct((N, F), jnp.float32),
    )(node_features, parts)
    return out
